# baseline (speedup 1.0000x reference)
import jax
import jax.numpy as jnp
from jax import lax
from jax.experimental import pallas as pl
from jax.experimental.pallas import tpu as pltpu
from jax.experimental.pallas import tpu_sc as plsc

N_NODES = 10000
N_EDGES = 320000
D = 128

NC = 2
NS = 16
NW = NC * NS

EPT = N_EDGES // NW
C = 40
NCH = EPT // C
NPAD = 10240
RPS = NPAD // NS
LANES = 16
DV = D // LANES

NBUF = 4
NSB = 2
NMETA = 8


def _sc_body(h_hbm, src_hbm, dst_hbm, w_hbm, out_hbm,
             b0, b1, b2, b3, sb0, sb1, srcr, dstr, wr,
             agg_sh,
             g0, g1, g2, g3, s0, s1,
             p0, p1, p2, p3, p4, p5, p6, p7):
  cid = lax.axis_index("c")
  sid = lax.axis_index("s")
  wid = cid * NS + sid
  bufs = (b0, b1, b2, b3)
  sbufs = (sb0, sb1)
  gsem = (g0, g1, g2, g3)
  ssem = (s0, s1)
  msem = (p0, p1, p2, p3, p4, p5, p6, p7)

  def _meta_copies(t, i):
    off = (wid * NCH + t) * C
    return (
        pltpu.make_async_copy(src_hbm.at[pl.ds(off, C)], srcr.at[i],
                              msem[i]),
        pltpu.make_async_copy(dst_hbm.at[pl.ds(off, C)], dstr.at[i],
                              msem[i]),
        pltpu.make_async_copy(w_hbm.at[pl.ds(off, C)], wr.at[i], msem[i]),
    )

  def _issue_meta(t, i):
    for cp in _meta_copies(t, i):
      cp.start()

  def _wait_meta(t, i):
    for cp in _meta_copies(t, i):
      cp.wait()

  def _issue_gather(i):
    k = i % NBUF
    pltpu.async_copy(h_hbm.at[srcr.at[i]], bufs[k], gsem[k])

  def _wait_gather(i):
    k = i % NBUF
    pltpu.make_async_copy(h_hbm.at[srcr.at[i]], bufs[k], gsem[k]).wait()

  def _issue_scatter(i):
    k = i % NSB
    pltpu.async_copy(sbufs[k], agg_sh.at[dstr.at[i]], ssem[k], add=True)

  def _wait_scatter(i):
    k = i % NSB
    pltpu.make_async_copy(sbufs[k], agg_sh.at[dstr.at[i]], ssem[k]).wait()

  def _scale(i):
    buf = bufs[i % NBUF]
    sbuf = sbufs[i % NSB]
    i16 = jnp.full((LANES,), i, jnp.int32)

    @plsc.parallel_loop(0, C, unroll=5)
    def _edge(e):
      w16 = plsc.load_gather(wr, [i16, jnp.full((LANES,), e, jnp.int32)])
      for j in range(DV):
        sl = pl.ds(j * LANES, LANES)
        sbuf[e, sl] = buf[e, sl] * w16

  def _chunk(t, j, c_lo, c_hi):
    _wait_gather(j)
    if c_lo >= NSB:
      _wait_scatter((j + NMETA - NSB) % NMETA)
    _scale(j)
    _issue_scatter(j)
    if c_hi + 3 <= NCH - 1:
      _wait_meta(t + 3, (j + 3) % NMETA)
      _issue_gather((j + 3) % NMETA)
    if c_hi + 5 <= NCH - 1:
      _issue_meta(t + 5, (j + 5) % NMETA)

  for t in range(5):
    _issue_meta(t, t)

  zero16 = jnp.zeros((LANES,), jnp.float32)

  def _zero_row(r, carry):
    for j in range(DV):
      sb0[r, pl.ds(j * LANES, LANES)] = zero16
    return carry

  lax.fori_loop(0, C, _zero_row, 0)
  for k in range(RPS // C):
    pltpu.sync_copy(sb0, agg_sh.at[pl.ds(sid * RPS + k * C, C)])
  plsc.subcore_barrier()

  for t in range(3):
    _wait_meta(t, t)
    _issue_gather(t)

  for t in range(NMETA):
    _chunk(t, t, t, t)

  def _round(r, carry):
    base = r * NMETA
    for j in range(NMETA):
      _chunk(base + j, j, NMETA, NCH - 1 - 10)
    return carry

  lax.fori_loop(1, (NCH - 10) // NMETA, _round, 0)

  for t in range(NCH - 10, NCH):
    _chunk(t, t % NMETA, t, t)

  for t in range(NCH - NSB, NCH):
    _wait_scatter(t % NMETA)
  plsc.subcore_barrier()
  pltpu.sync_copy(agg_sh.at[pl.ds(sid * RPS, RPS)],
                  out_hbm.at[cid, pl.ds(sid * RPS, RPS)])


_sc_spmm = pl.kernel(
    _sc_body,
    out_type=jax.ShapeDtypeStruct((NC, NPAD, D), jnp.float32),
    mesh=plsc.VectorSubcoreMesh(core_axis_name="c", subcore_axis_name="s"),
    scratch_types=(
        [pltpu.VMEM((C, D), jnp.float32) for _ in range(NBUF)]
        + [pltpu.VMEM((C, D), jnp.float32) for _ in range(NSB)]
        + [pltpu.VMEM((NMETA, C), jnp.int32) for _ in range(2)]
        + [pltpu.VMEM((NMETA, C), jnp.float32)]
        + [pltpu.VMEM_SHARED((NPAD, D), jnp.float32)]
        + [pltpu.SemaphoreType.DMA] * (NBUF + NSB + NMETA)
    ),
    compiler_params=pltpu.CompilerParams(needs_layout_passes=False),
)


def _tc_linear_body(p0_ref, p1_ref, w_ref, b_ref, o_ref):
  x = p0_ref[...] + p1_ref[...]
  o_ref[...] = lax.dot_general(
      x, w_ref[...], (((1,), (1,)), ((), ())),
      preferred_element_type=jnp.float32) + b_ref[...]


_ROWS_BLK = 1000


def _tc_linear(p0, p1, W, b2d):
  grid = (N_NODES // _ROWS_BLK,)
  return pl.pallas_call(
      _tc_linear_body,
      grid=grid,
      in_specs=[
          pl.BlockSpec((_ROWS_BLK, D), lambda i: (i, 0)),
          pl.BlockSpec((_ROWS_BLK, D), lambda i: (i, 0)),
          pl.BlockSpec((D, D), lambda i: (0, 0)),
          pl.BlockSpec((1, D), lambda i: (0, 0)),
      ],
      out_specs=pl.BlockSpec((_ROWS_BLK, D), lambda i: (i, 0)),
      out_shape=jax.ShapeDtypeStruct((N_NODES, D), jnp.float32),
  )(p0, p1, W, b2d)


@jax.jit
def kernel(h, edge_index, edge_weight, W, b):
  ei = edge_index.astype(jnp.int32)
  partials = _sc_spmm(h, ei[0], ei[1], edge_weight)
  return _tc_linear(partials[0], partials[1], W, b.reshape(1, D))


# speedup vs baseline: 1.1268x; 1.1268x over previous
import jax
import jax.numpy as jnp
from jax import lax
from jax.experimental import pallas as pl
from jax.experimental.pallas import tpu as pltpu
from jax.experimental.pallas import tpu_sc as plsc

N_NODES = 10000
N_EDGES = 320000
D = 128

NC = 2
NS = 16
NW = NC * NS

EPT = N_EDGES // NW
C = 80
NCH = EPT // C
NPAD = 10240
RPS = NPAD // NS
LANES = 16
DV = D // LANES

NBUF = 4
NMETA = 8


def _sc_body(h_hbm, src_hbm, dst_hbm, w_hbm, out_hbm,
             b0, b1, b2, b3, srcr, dstr, wr,
             agg_sh,
             g0, g1, g2, g3, s0, s1, s2, s3,
             p0, p1, p2, p3, p4, p5, p6, p7):
  cid = lax.axis_index("c")
  sid = lax.axis_index("s")
  wid = cid * NS + sid
  bufs = (b0, b1, b2, b3)
  gsem = (g0, g1, g2, g3)
  ssem = (s0, s1, s2, s3)
  msem = (p0, p1, p2, p3, p4, p5, p6, p7)

  def _meta_copies(t, i):
    off = (wid * NCH + t) * C
    return (
        pltpu.make_async_copy(src_hbm.at[pl.ds(off, C)], srcr.at[i],
                              msem[i]),
        pltpu.make_async_copy(dst_hbm.at[pl.ds(off, C)], dstr.at[i],
                              msem[i]),
        pltpu.make_async_copy(w_hbm.at[pl.ds(off, C)], wr.at[i], msem[i]),
    )

  def _issue_meta(t, i):
    for cp in _meta_copies(t, i):
      cp.start()

  def _wait_meta(t, i):
    for cp in _meta_copies(t, i):
      cp.wait()

  def _issue_gather(i):
    k = i % NBUF
    pltpu.async_copy(h_hbm.at[srcr.at[i]], bufs[k], gsem[k])

  def _wait_gather(i):
    k = i % NBUF
    pltpu.make_async_copy(h_hbm.at[srcr.at[i]], bufs[k], gsem[k]).wait()

  def _issue_scatter(i):
    k = i % NBUF
    pltpu.async_copy(bufs[k], agg_sh.at[dstr.at[i]], ssem[k], add=True)

  def _wait_scatter(i):
    k = i % NBUF
    pltpu.make_async_copy(bufs[k], agg_sh.at[dstr.at[i]], ssem[k]).wait()

  def _scale(i):
    buf = bufs[i % NBUF]
    i16 = jnp.full((LANES,), i, jnp.int32)

    @plsc.parallel_loop(0, C, unroll=5)
    def _edge(e):
      w16 = plsc.load_gather(wr, [i16, jnp.full((LANES,), e, jnp.int32)])
      for j in range(DV):
        sl = pl.ds(j * LANES, LANES)
        buf[e, sl] = buf[e, sl] * w16

  def _chunk(t, j, c_lo, c_hi):
    _wait_gather(j)
    _scale(j)
    _issue_scatter(j)
    if c_hi + 3 <= NCH - 1:
      if c_lo >= 1:
        _wait_scatter((j + NMETA - 1) % NMETA)
      _wait_meta(t + 3, (j + 3) % NMETA)
      _issue_gather((j + 3) % NMETA)
    if c_hi + 5 <= NCH - 1:
      _issue_meta(t + 5, (j + 5) % NMETA)

  for t in range(5):
    _issue_meta(t, t)

  zero16 = jnp.zeros((LANES,), jnp.float32)

  def _zero_row(r, carry):
    for j in range(DV):
      b0[r, pl.ds(j * LANES, LANES)] = zero16
    return carry

  lax.fori_loop(0, C, _zero_row, 0)
  for k in range(RPS // C):
    pltpu.sync_copy(b0, agg_sh.at[pl.ds(sid * RPS + k * C, C)])
  plsc.subcore_barrier()

  for t in range(3):
    _wait_meta(t, t)
    _issue_gather(t)

  for t in range(NMETA):
    _chunk(t, t, t, t)

  def _round(r, carry):
    base = r * NMETA
    for j in range(NMETA):
      _chunk(base + j, j, NMETA, NCH - 1 - 5)
    return carry

  lax.fori_loop(1, (NCH - 5) // NMETA, _round, 0)

  for t in range(NCH - 5, NCH):
    _chunk(t, t % NMETA, t, t)

  for t in range(NCH - 4, NCH):
    _wait_scatter(t % NMETA)
  plsc.subcore_barrier()
  pltpu.sync_copy(agg_sh.at[pl.ds(sid * RPS, RPS)],
                  out_hbm.at[cid, pl.ds(sid * RPS, RPS)])


_sc_spmm = pl.kernel(
    _sc_body,
    out_type=jax.ShapeDtypeStruct((NC, NPAD, D), jnp.float32),
    mesh=plsc.VectorSubcoreMesh(core_axis_name="c", subcore_axis_name="s"),
    scratch_types=(
        [pltpu.VMEM((C, D), jnp.float32) for _ in range(NBUF)]
        + [pltpu.VMEM((NMETA, C), jnp.int32) for _ in range(2)]
        + [pltpu.VMEM((NMETA, C), jnp.float32)]
        + [pltpu.VMEM_SHARED((NPAD, D), jnp.float32)]
        + [pltpu.SemaphoreType.DMA] * (NBUF + NBUF + NMETA)
    ),
    compiler_params=pltpu.CompilerParams(needs_layout_passes=False),
)


def _tc_linear_body(p0_ref, p1_ref, w_ref, b_ref, o_ref):
  x = p0_ref[...] + p1_ref[...]
  o_ref[...] = lax.dot_general(
      x, w_ref[...], (((1,), (1,)), ((), ())),
      preferred_element_type=jnp.float32) + b_ref[...]


_ROWS_BLK = 1000


def _tc_linear(p0, p1, W, b2d):
  grid = (N_NODES // _ROWS_BLK,)
  return pl.pallas_call(
      _tc_linear_body,
      grid=grid,
      in_specs=[
          pl.BlockSpec((_ROWS_BLK, D), lambda i: (i, 0)),
          pl.BlockSpec((_ROWS_BLK, D), lambda i: (i, 0)),
          pl.BlockSpec((D, D), lambda i: (0, 0)),
          pl.BlockSpec((1, D), lambda i: (0, 0)),
      ],
      out_specs=pl.BlockSpec((_ROWS_BLK, D), lambda i: (i, 0)),
      out_shape=jax.ShapeDtypeStruct((N_NODES, D), jnp.float32),
  )(p0, p1, W, b2d)


@jax.jit
def kernel(h, edge_index, edge_weight, W, b):
  ei = edge_index.astype(jnp.int32)
  partials = _sc_spmm(h, ei[0], ei[1], edge_weight)
  return _tc_linear(partials[0], partials[1], W, b.reshape(1, D))
